# baseline (speedup 1.0000x reference)
import jax
import jax.numpy as jnp
from jax.experimental import pallas as pl
from jax.experimental.pallas import tpu as pltpu
import jax.experimental.pallas.tpu_sc as plsc

N_EXPERTS = 4
HIDDEN = 16
TOP_K = 2
NUM_QUERIES = 100
NUM_CLASSES = 92
QC = NUM_QUERIES * NUM_CLASSES
FEAT = 588
HALF = 2304
LGW = 4 * HALF

POOL_G = 4
POOL_ROWS = 10752 // POOL_G


def _pool_body(x_ref, o_ref):
    X = x_ref[...]
    S = jnp.sum(X.reshape(POOL_ROWS // 16, 16, 224), axis=1)
    r224 = jax.lax.broadcasted_iota(jnp.int32, (224, 14), 0)
    c14 = jax.lax.broadcasted_iota(jnp.int32, (224, 14), 1)
    Bm = jnp.where(r224 // 16 == c14, 1.0 / 256.0, 0.0).astype(jnp.float32)
    o_ref[...] = jnp.dot(S, Bm, preferred_element_type=jnp.float32)


def _expert_body(f_ref, wla_ref, wlb_ref, wlc_ref, wld_ref, bl_ref,
                 lg_ref, c0_ref):
    i = pl.program_id(0)
    B = f_ref.shape[0]
    part_row = None
    for half, wref in enumerate((wla_ref, wlb_ref, wlc_ref, wld_ref)):
        res = jnp.dot(f_ref[...], wref[0],
                      preferred_element_type=jnp.float32)
        lg_ref[pl.ds(i * B, B), pl.ds(half * HALF, HALF)] = res
        lane = jax.lax.broadcasted_iota(jnp.int32, (1, HALF), 1) + half * HALF
        mrow = jnp.where((lane % NUM_CLASSES == 0) & (lane < QC),
                         1.0, 0.0)
        p = jax.lax.dot_general(mrow, res, (((1,), (1,)), ((), ())),
                                preferred_element_type=jnp.float32)
        part_row = p if part_row is None else part_row + p
    row = jax.lax.broadcasted_iota(jnp.int32, (8, B), 0)
    upd = jnp.where(row == i, part_row, 0.0)

    @pl.when(i == 0)
    def _init():
        bl2d = bl_ref[...]
        blane = jax.lax.broadcasted_iota(jnp.int32, (1, QC), 1)
        bmask = jnp.where(blane % NUM_CLASSES == 0, 1.0, 0.0)
        bvec = jax.lax.dot_general(bl2d, bmask, (((1,), (1,)), ((), ())),
                                   preferred_element_type=jnp.float32)
        bias = jnp.concatenate(
            [bvec * jnp.ones((1, B), jnp.float32),
             jnp.zeros((8 - N_EXPERTS, B), jnp.float32)], axis=0)
        c0_ref[...] = upd + bias

    @pl.when(i > 0)
    def _acc():
        c0_ref[...] += upd


def _route_sc_body(c0_hbm, w1_hbm, b1_hbm, w2_hbm, b2_hbm,
                   ep_hbm, nw_hbm, ti_hbm, fp_hbm,
                   c0_v, w1_v, b1_v, w2_v, b2_v, out_v, sem_in, sem_out):
    c = jax.lax.axis_index("c")
    s = jax.lax.axis_index("s")

    @pl.when((c == 0) & (s == 0))
    def _():
        pltpu.make_async_copy(c0_hbm, c0_v, sem_in).start()
        pltpu.make_async_copy(w1_hbm, w1_v, sem_in).start()
        pltpu.make_async_copy(b1_hbm, b1_v, sem_in).start()
        pltpu.make_async_copy(w2_hbm, w2_v, sem_in).start()
        pltpu.make_async_copy(b2_hbm, b2_v, sem_in).start()
        pltpu.make_async_copy(c0_hbm, c0_v, sem_in).wait()
        pltpu.make_async_copy(w1_hbm, w1_v, sem_in).wait()
        pltpu.make_async_copy(b1_hbm, b1_v, sem_in).wait()
        pltpu.make_async_copy(w2_hbm, w2_v, sem_in).wait()
        pltpu.make_async_copy(b2_hbm, b2_v, sem_in).wait()

        inv_q = 1.0 / NUM_QUERIES
        ep = []
        for e in range(N_EXPERTS):
            x = c0_v[e] * inv_q
            ep.append(1.0 / (1.0 + jnp.exp(-x)))
        h = []
        for j in range(HIDDEN):
            acc = b1_v[j]
            for e in range(N_EXPERTS):
                acc = acc + ep[e] * w1_v[e * HIDDEN + j]
            h.append(jnp.maximum(acc, 0.0))
        z = []
        for e in range(N_EXPERTS):
            acc = b2_v[e]
            for j in range(HIDDEN):
                acc = acc + h[j] * w2_v[j * N_EXPERTS + e]
            z.append(acc)
        zmax = jnp.maximum(jnp.maximum(z[0], z[1]), jnp.maximum(z[2], z[3]))
        ez = [jnp.exp(zz - zmax) for zz in z]
        ssum = ez[0] + ez[1] + ez[2] + ez[3]
        w = [e_ / ssum for e_ in ez]

        m1 = jnp.maximum(jnp.maximum(w[0], w[1]), jnp.maximum(w[2], w[3]))
        i1 = jnp.full((16,), N_EXPERTS, jnp.int32)
        for e in range(N_EXPERTS - 1, -1, -1):
            i1 = jnp.where(w[e] == m1, jnp.full((16,), e, jnp.int32), i1)
        wx = [jnp.where(i1 == e, jnp.full((16,), -1.0, jnp.float32), w[e])
              for e in range(N_EXPERTS)]
        m2 = jnp.maximum(jnp.maximum(wx[0], wx[1]), jnp.maximum(wx[2], wx[3]))
        i2 = jnp.full((16,), N_EXPERTS, jnp.int32)
        for e in range(N_EXPERTS - 1, -1, -1):
            i2 = jnp.where(wx[e] == m2, jnp.full((16,), e, jnp.int32), i2)

        nw = [jnp.where((i1 == e) | (i2 == e), w[e],
                        jnp.zeros((16,), jnp.float32))
              for e in range(N_EXPERTS)]
        nsum = nw[0] + nw[1] + nw[2] + nw[3] + 1e-8
        nw = [n / nsum for n in nw]
        fp = nw[0] * ep[0] + nw[1] * ep[1] + nw[2] * ep[2] + nw[3] * ep[3]

        for e in range(N_EXPERTS):
            out_v[e] = ep[e]
            out_v[N_EXPERTS + e] = nw[e]
        out_v[2 * N_EXPERTS] = jax.lax.bitcast_convert_type(i1, jnp.float32)
        out_v[2 * N_EXPERTS + 1] = jax.lax.bitcast_convert_type(i2, jnp.float32)
        out_v[2 * N_EXPERTS + 2] = fp

        pltpu.make_async_copy(out_v.at[pl.ds(0, N_EXPERTS)],
                              ep_hbm, sem_out).start()
        pltpu.make_async_copy(out_v.at[pl.ds(N_EXPERTS, N_EXPERTS)],
                              nw_hbm, sem_out).start()
        pltpu.make_async_copy(out_v.at[pl.ds(2 * N_EXPERTS, TOP_K)],
                              ti_hbm, sem_out).start()
        pltpu.make_async_copy(out_v.at[pl.ds(2 * N_EXPERTS + 2, 1)],
                              fp_hbm, sem_out).start()
        pltpu.make_async_copy(out_v.at[pl.ds(0, N_EXPERTS)],
                              ep_hbm, sem_out).wait()
        pltpu.make_async_copy(out_v.at[pl.ds(N_EXPERTS, N_EXPERTS)],
                              nw_hbm, sem_out).wait()
        pltpu.make_async_copy(out_v.at[pl.ds(2 * N_EXPERTS, TOP_K)],
                              ti_hbm, sem_out).wait()
        pltpu.make_async_copy(out_v.at[pl.ds(2 * N_EXPERTS + 2, 1)],
                              fp_hbm, sem_out).wait()


def _combine_body(lg_ref, nwt_ref, bl_ref, f_ref, wb_ref, bb_ref,
                  comb_ref, box_ref):
    B = f_ref.shape[0]
    nw = nwt_ref[...].T
    comb = jnp.dot(nw, bl_ref[...], preferred_element_type=jnp.float32)
    for e in range(N_EXPERTS):
        comb = comb + nw[:, e:e + 1] * lg_ref[e * B:(e + 1) * B, 0:QC]
    comb_ref[...] = comb
    bx = jnp.dot(f_ref[...], wb_ref[...], preferred_element_type=jnp.float32)
    box_ref[...] = jax.nn.sigmoid(bx + bb_ref[...])


def kernel(pixel_values, Wl, bl, Wb, bb, W1, b1, W2, b2):
    B = pixel_values.shape[0]
    BC = B * 3

    x = pixel_values.reshape(BC * 224, 224)
    pooled = pl.pallas_call(
        _pool_body,
        grid=(POOL_G,),
        in_specs=[pl.BlockSpec((POOL_ROWS, 224), lambda i: (i, 0))],
        out_specs=pl.BlockSpec((POOL_ROWS // 16, 14), lambda i: (i, 0)),
        out_shape=jax.ShapeDtypeStruct((BC * 14, 14), jnp.float32),
    )(x)
    f = pooled.reshape(B, FEAT)

    def _wl_map(k):
        return lambda i: (i, 0, k)

    const2 = lambda i: (0, 0)

    lg, c0 = pl.pallas_call(
        _expert_body,
        grid=(N_EXPERTS,),
        in_specs=[
            pl.BlockSpec((B, FEAT), const2),
            pl.BlockSpec((1, FEAT, HALF), _wl_map(0)),
            pl.BlockSpec((1, FEAT, HALF), _wl_map(1)),
            pl.BlockSpec((1, FEAT, HALF), _wl_map(2)),
            pl.BlockSpec((1, FEAT, HALF), _wl_map(3)),
            pl.BlockSpec((N_EXPERTS, QC), const2),
        ],
        out_specs=[
            pl.BlockSpec((N_EXPERTS * B, LGW), const2),
            pl.BlockSpec((8, B), const2),
        ],
        out_shape=[
            jax.ShapeDtypeStruct((N_EXPERTS * B, LGW), jnp.float32),
            jax.ShapeDtypeStruct((8, B), jnp.float32),
        ],
    )(f, Wl, Wl, Wl, Wl, bl)

    route = pl.kernel(
        _route_sc_body,
        out_type=[
            jax.ShapeDtypeStruct((N_EXPERTS, B), jnp.float32),
            jax.ShapeDtypeStruct((N_EXPERTS, B), jnp.float32),
            jax.ShapeDtypeStruct((TOP_K, B), jnp.float32),
            jax.ShapeDtypeStruct((1, B), jnp.float32),
        ],
        mesh=plsc.VectorSubcoreMesh(core_axis_name="c", subcore_axis_name="s"),
        scratch_types=[
            pltpu.VMEM((8, B), jnp.float32),
            pltpu.VMEM((N_EXPERTS * HIDDEN, B), jnp.float32),
            pltpu.VMEM((HIDDEN, B), jnp.float32),
            pltpu.VMEM((HIDDEN * N_EXPERTS, B), jnp.float32),
            pltpu.VMEM((N_EXPERTS, B), jnp.float32),
            pltpu.VMEM((2 * N_EXPERTS + 3, B), jnp.float32),
            pltpu.SemaphoreType.DMA,
            pltpu.SemaphoreType.DMA,
        ],
    )
    W1b = jnp.broadcast_to(W1.reshape(N_EXPERTS, HIDDEN, 1),
                           (N_EXPERTS, HIDDEN, B)).reshape(N_EXPERTS * HIDDEN, B)
    b1b = jnp.broadcast_to(b1.reshape(HIDDEN, 1), (HIDDEN, B))
    W2b = jnp.broadcast_to(W2.reshape(HIDDEN, N_EXPERTS, 1),
                           (HIDDEN, N_EXPERTS, B)).reshape(HIDDEN * N_EXPERTS, B)
    b2b = jnp.broadcast_to(b2.reshape(N_EXPERTS, 1), (N_EXPERTS, B))
    ept, nwt, tit_f, fp = route(c0, W1b, b1b, W2b, b2b)
    ti = jax.lax.bitcast_convert_type(tit_f, jnp.int32)

    const0 = lambda: (0, 0)
    comb, box = pl.pallas_call(
        _combine_body,
        in_specs=[
            pl.BlockSpec((N_EXPERTS * B, LGW), const0),
            pl.BlockSpec((N_EXPERTS, B), const0),
            pl.BlockSpec((N_EXPERTS, QC), const0),
            pl.BlockSpec((B, FEAT), const0),
            pl.BlockSpec((FEAT, 4 * NUM_QUERIES), const0),
            pl.BlockSpec((1, 4 * NUM_QUERIES), const0),
        ],
        out_specs=[
            pl.BlockSpec((B, QC), const0),
            pl.BlockSpec((B, 4 * NUM_QUERIES), const0),
        ],
        out_shape=[
            jax.ShapeDtypeStruct((B, QC), jnp.float32),
            jax.ShapeDtypeStruct((B, 4 * NUM_QUERIES), jnp.float32),
        ],
    )(lg, nwt, bl, f, Wb[0], bb[0].reshape(1, 4 * NUM_QUERIES))

    combined_logits = comb.reshape(B, NUM_QUERIES, NUM_CLASSES)
    pred_boxes = box.reshape(B, NUM_QUERIES, 4)
    return (combined_logits, pred_boxes, fp.reshape(B), nwt.T, ept.T, ti.T)
